# Initial kernel scaffold; baseline (speedup 1.0000x reference)
import jax
import jax.numpy as jnp
from jax.experimental import pallas as pl


def kernel(x, edge_index, edge_weight, W1, b1, W2, b2, Wadv, badv, Wv1, bv1, Wv2, bv2, Wv3, bv3):
    raise NotImplementedError("write your pallas kernel here")


# speedup vs baseline: 2.9871x; 2.9871x over previous
import functools

import jax
import jax.numpy as jnp
from jax import lax
from jax.experimental import pallas as pl
from jax.experimental.pallas import tpu as pltpu
from jax.experimental.pallas import tpu_sc as plsc

N = 10000
NP = 10240
E = 320000
D = 128
NC = 2
NS = 16
NW = NC * NS
K = 128
EPP = 335872
CH = 640
CHL = N - CH * (NS - 1)

_MESH = dict(core_axis_name="c", subcore_axis_name="s")


def _chunk_copy(s, mk_src, mk_dst):
    @pl.when(s < NS - 1)
    def _():
        pltpu.sync_copy(mk_src(s * CH, CH), mk_dst(s * CH, CH))

    @pl.when(s == NS - 1)
    def _():
        pltpu.sync_copy(mk_src((NS - 1) * CH, CHL),
                        mk_dst((NS - 1) * CH, CHL))


def _sc_deg_body(dst_hbm, w_hbm, out_hbm, deg_v, dst_v, w_v):
    c = lax.axis_index("c")
    s = lax.axis_index("s")
    tid = c * NS + s

    def zero_body(i, carry):
        deg_v[i, :] = jnp.zeros((16,), jnp.float32)
        return carry

    lax.fori_loop(0, NP // 16, zero_body, 0)

    edges_per = EPP // NW
    nb = edges_per // K

    def body(b, carry):
        base = tid * edges_per + b * K
        pltpu.sync_copy(dst_hbm.at[pl.ds(base, K)], dst_v)
        pltpu.sync_copy(w_hbm.at[pl.ds(base, K)], w_v)
        for j in range(K // 16):
            sl = pl.ds(j * 16, 16)
            d16 = dst_v[sl]
            plsc.addupdate_scatter(
                deg_v, [lax.shift_right_logical(d16, 4),
                        lax.bitwise_and(d16, 15)], w_v[sl])
        return carry

    lax.fori_loop(0, nb, body, 0)
    pltpu.sync_copy(deg_v, out_hbm.at[pl.ds(tid * (NP // 16), NP // 16)])


def _sc_deg(dst, w):
    kern = functools.partial(
        pl.kernel,
        out_type=jax.ShapeDtypeStruct((NW * NP // 16, 16), jnp.float32),
        mesh=plsc.VectorSubcoreMesh(**_MESH),
        compiler_params=pltpu.CompilerParams(needs_layout_passes=False),
        scratch_types=[
            pltpu.VMEM((NP // 16, 16), jnp.float32),
            pltpu.VMEM((K,), jnp.int32),
            pltpu.VMEM((K,), jnp.float32),
        ],
    )(_sc_deg_body)
    return kern(dst, w)


def _sc_norm_body(dinv_hbm, src_hbm, dst_hbm, w_hbm, out_hbm,
                  dinv_v, src_v, dst_v, w_v, norm_v):
    c = lax.axis_index("c")
    s = lax.axis_index("s")
    tid = c * NS + s
    pltpu.sync_copy(dinv_hbm, dinv_v)

    edges_per = EPP // NW
    nb = edges_per // K

    def body(b, carry):
        base = tid * edges_per + b * K
        pltpu.sync_copy(src_hbm.at[pl.ds(base, K)], src_v)
        pltpu.sync_copy(dst_hbm.at[pl.ds(base, K)], dst_v)
        pltpu.sync_copy(w_hbm.at[pl.ds(base, K)], w_v)
        for j in range(K // 16):
            sl = pl.ds(j * 16, 16)
            a = plsc.load_gather(dinv_v, [src_v[sl]])
            b_ = plsc.load_gather(dinv_v, [dst_v[sl]])
            norm_v[sl] = a * b_ * w_v[sl]
        pltpu.sync_copy(norm_v, out_hbm.at[pl.ds(base, K)])
        return carry

    lax.fori_loop(0, nb, body, 0)


def _sc_norm(dinv, src, dst, w):
    kern = functools.partial(
        pl.kernel,
        out_type=jax.ShapeDtypeStruct((EPP,), jnp.float32),
        mesh=plsc.VectorSubcoreMesh(**_MESH),
        compiler_params=pltpu.CompilerParams(needs_layout_passes=False),
        scratch_types=[
            pltpu.VMEM((NP,), jnp.float32),
            pltpu.VMEM((K,), jnp.int32),
            pltpu.VMEM((K,), jnp.int32),
            pltpu.VMEM((K,), jnp.float32),
            pltpu.VMEM((K,), jnp.float32),
        ],
    )(_sc_norm_body)
    return kern(dinv, src, dst, w)


def _sc_conv_body(chan_split, u_hbm, src_hbm, dst_hbm, norm_hbm, out_hbm,
                  acc, idx_a, idx_b, dst_a, dst_b, w_a, w_b,
                  rows_a, rows_b, sem_a, sem_b):
    c = lax.axis_index("c")
    s = lax.axis_index("s")
    if chan_split:
        row_off = c * N
        edges_per = EPP // NS
        chunk0 = s * edges_per
    else:
        row_off = 0
        edges_per = EPP // NW
        chunk0 = (c * NS + s) * edges_per
    nb = edges_per // K

    _chunk_copy(s, lambda o, n: u_hbm.at[pl.ds(row_off + o, n)],
                lambda o, n: acc.at[pl.ds(o, n)])
    plsc.subcore_barrier()

    def load_batch(b, idx_v, dst_v, w_v, sem, rows_v):
        base = chunk0 + b * K
        pltpu.sync_copy(src_hbm.at[pl.ds(base, K)], idx_v)
        pltpu.sync_copy(dst_hbm.at[pl.ds(base, K)], dst_v)
        pltpu.sync_copy(norm_hbm.at[pl.ds(base, K)], w_v)
        if chan_split:
            for g in range(K // 16):
                sl = pl.ds(g * 16, 16)
                idx_v[sl] = idx_v[sl] + row_off
        return pltpu.async_copy(u_hbm.at[idx_v], rows_v, sem)

    def process_batch(dst_v, w_v, rows_v):
        def grp_body(j, carry):
            wv = w_v[pl.ds(j * 16, 16)]
            for l in range(16):
                wk = wv[l]
                k = j * 16 + l
                for g in range(D // 16):
                    sl = pl.ds(g * 16, 16)
                    rows_v[k, sl] = rows_v[k, sl] * wk
            return carry

        lax.fori_loop(0, K // 16, grp_body, 0)
        pltpu.sync_copy(rows_v, acc.at[dst_v], add=True)

    load_batch(0, idx_a, dst_a, w_a, sem_a, rows_a)

    def body(i, carry):
        b = i * 2
        nxt = load_batch(b + 1, idx_b, dst_b, w_b, sem_b, rows_b)
        pltpu.make_async_copy(u_hbm.at[idx_a], rows_a, sem_a).wait()
        process_batch(dst_a, w_a, rows_a)

        @pl.when(b + 2 < nb)
        def _():
            load_batch(b + 2, idx_a, dst_a, w_a, sem_a, rows_a)

        nxt.wait()
        process_batch(dst_b, w_b, rows_b)
        return carry

    lax.fori_loop(0, nb // 2, body, 0)
    plsc.subcore_barrier()
    _chunk_copy(s, lambda o, n: acc.at[pl.ds(o, n)],
                lambda o, n: out_hbm.at[c, pl.ds(o, n)])


def _sc_conv(u, src, dst, norm, chan_split):
    kern = functools.partial(
        pl.kernel,
        out_type=jax.ShapeDtypeStruct((NC, N, D), jnp.float32),
        mesh=plsc.VectorSubcoreMesh(**_MESH),
        compiler_params=pltpu.CompilerParams(needs_layout_passes=False),
        scratch_types=[
            pltpu.VMEM_SHARED((N, D), jnp.float32),
            pltpu.VMEM((K,), jnp.int32),
            pltpu.VMEM((K,), jnp.int32),
            pltpu.VMEM((K,), jnp.int32),
            pltpu.VMEM((K,), jnp.int32),
            pltpu.VMEM((K,), jnp.float32),
            pltpu.VMEM((K,), jnp.float32),
            pltpu.VMEM((K, D), jnp.float32),
            pltpu.VMEM((K, D), jnp.float32),
            pltpu.SemaphoreType.DMA,
            pltpu.SemaphoreType.DMA,
        ],
    )(functools.partial(_sc_conv_body, chan_split))
    return kern(u, src, dst, norm)


NB = 1000


def _dinv_body(degp_ref, dinv_ref):
    deg = jnp.sum(degp_ref[...], axis=0, keepdims=True)
    dinv_ref[...] = lax.rsqrt(jnp.maximum(deg, 1.0))


def _tc_dinv(degp):
    return pl.pallas_call(
        _dinv_body,
        grid=(1,),
        in_specs=[pl.BlockSpec((NW, NP), lambda i: (0, 0))],
        out_specs=pl.BlockSpec((1, NP), lambda i: (0, 0)),
        out_shape=jax.ShapeDtypeStruct((1, NP), jnp.float32),
    )(degp)


def _mm1_body(x_ref, w1_ref, u1_ref):
    u1_ref[...] = jnp.dot(x_ref[...], w1_ref[...],
                          preferred_element_type=jnp.float32)


def _tc_mm1(xs, w1):
    return pl.pallas_call(
        _mm1_body,
        grid=(N // NB,),
        in_specs=[
            pl.BlockSpec((NB, D), lambda i: (i, 0)),
            pl.BlockSpec((D, D), lambda i: (0, 0)),
        ],
        out_specs=pl.BlockSpec((NB, D), lambda i: (i, 0)),
        out_shape=jax.ShapeDtypeStruct((N, D), jnp.float32),
    )(xs, w1)


def _mm2_body(acc1_ref, u1_ref, b1_ref, w2_ref, u2_ref):
    h = jnp.maximum(acc1_ref[0] + acc1_ref[1] - 2.0 * u1_ref[...]
                    + b1_ref[...], 0.0)
    pre = jnp.dot(h, w2_ref[...], preferred_element_type=jnp.float32)
    u2_ref[0] = pre[:, :128]
    u2_ref[1] = pre[:, 128:]


def _tc_mm2(acc1, u1, b1, w2):
    return pl.pallas_call(
        _mm2_body,
        grid=(N // NB,),
        in_specs=[
            pl.BlockSpec((NC, NB, D), lambda i: (0, i, 0)),
            pl.BlockSpec((NB, D), lambda i: (i, 0)),
            pl.BlockSpec((1, D), lambda i: (0, 0)),
            pl.BlockSpec((D, 256), lambda i: (0, 0)),
        ],
        out_specs=pl.BlockSpec((NC, NB, 128), lambda i: (0, i, 0)),
        out_shape=jax.ShapeDtypeStruct((NC, N, 128), jnp.float32),
    )(acc1, u1, b1, w2)


def _h2_body(acc2_ref, u2_ref, b2_ref, h2_ref):
    h2_ref[:, :128] = jnp.maximum(
        acc2_ref[0] - u2_ref[0] + b2_ref[:, :128], 0.0)
    h2_ref[:, 128:] = jnp.maximum(
        acc2_ref[1] - u2_ref[1] + b2_ref[:, 128:], 0.0)


def _tc_h2(acc2, u2, b2):
    return pl.pallas_call(
        _h2_body,
        grid=(N // NB,),
        in_specs=[
            pl.BlockSpec((NC, NB, 128), lambda i: (0, i, 0)),
            pl.BlockSpec((NC, NB, 128), lambda i: (0, i, 0)),
            pl.BlockSpec((1, 256), lambda i: (0, 0)),
        ],
        out_specs=pl.BlockSpec((NB, 256), lambda i: (i, 0)),
        out_shape=jax.ShapeDtypeStruct((N, 256), jnp.float32),
    )(acc2, u2, b2)


M = N * 256
MB = 10240
NSTEP = M // MB
A = 15


def _heads_body(flat_ref, wadv_ref, wv1_ref, badv_ref, bv1_ref,
                wv2_ref, bv2_ref, wv3_ref, bv3_ref, out_ref,
                adv_acc, v_acc):
    i = pl.program_id(0)

    @pl.when(i == 0)
    def _():
        adv_acc[...] = jnp.zeros_like(adv_acc)
        v_acc[...] = jnp.zeros_like(v_acc)

    fa = flat_ref[...]
    adv_acc[...] += jnp.dot(fa, wadv_ref[...],
                            preferred_element_type=jnp.float32)
    v_acc[...] += jnp.dot(fa, wv1_ref[...],
                          preferred_element_type=jnp.float32)

    @pl.when(i == NSTEP - 1)
    def _():
        adv = jnp.maximum(adv_acc[...] + badv_ref[...], 0.0)
        v = jnp.maximum(v_acc[...] + bv1_ref[...], 0.0)
        v = jnp.maximum(jnp.dot(v, wv2_ref[...],
                                preferred_element_type=jnp.float32)
                        + bv2_ref[...], 0.0)
        val = (jnp.dot(v, wv3_ref[...], preferred_element_type=jnp.float32)
               + bv3_ref[...])
        r = lax.broadcasted_iota(jnp.int32, (A, A), 0)
        col = lax.broadcasted_iota(jnp.int32, (A, A), 1)
        g = jnp.where(r // 5 == col // 5, 1.0 / 5.0, 0.0)
        m = jnp.dot(adv, g, preferred_element_type=jnp.float32)
        out_ref[...] = adv - m + val


def _tc_heads(flat, wadv, wv1, badv, bv1, wv2, bv2, wv3, bv3):
    return pl.pallas_call(
        _heads_body,
        grid=(NSTEP,),
        in_specs=[
            pl.BlockSpec((1, MB), lambda i: (0, i)),
            pl.BlockSpec((MB, A), lambda i: (i, 0)),
            pl.BlockSpec((MB, 64), lambda i: (i, 0)),
            pl.BlockSpec((1, A), lambda i: (0, 0)),
            pl.BlockSpec((1, 64), lambda i: (0, 0)),
            pl.BlockSpec((64, 64), lambda i: (0, 0)),
            pl.BlockSpec((1, 64), lambda i: (0, 0)),
            pl.BlockSpec((64, 1), lambda i: (0, 0)),
            pl.BlockSpec((1, 1), lambda i: (0, 0)),
        ],
        out_specs=pl.BlockSpec((1, A), lambda i: (0, 0)),
        out_shape=jax.ShapeDtypeStruct((1, A), jnp.float32),
        scratch_shapes=[
            pltpu.VMEM((1, A), jnp.float32),
            pltpu.VMEM((1, 64), jnp.float32),
        ],
    )(flat, wadv, wv1, badv, bv1, wv2, bv2, wv3, bv3)


def kernel(x, edge_index, edge_weight, W1, b1, W2, b2,
           Wadv, badv, Wv1, bv1, Wv2, bv2, Wv3, bv3):
    xs = x.reshape(N, D)
    loop = jnp.arange(N, dtype=jnp.int32)
    pad = EPP - E - N
    src = jnp.concatenate([edge_index[0].astype(jnp.int32), loop,
                           jnp.zeros((pad,), jnp.int32)])
    dst = jnp.concatenate([edge_index[1].astype(jnp.int32), loop,
                           jnp.zeros((pad,), jnp.int32)])
    w = jnp.concatenate([edge_weight.astype(jnp.float32),
                         jnp.ones((N,), jnp.float32),
                         jnp.zeros((pad,), jnp.float32)])

    degp = _sc_deg(dst, w)
    dinv = _tc_dinv(degp.reshape(NW, NP))
    norm = _sc_norm(dinv.reshape(NP), src, dst, w)
    u1 = _tc_mm1(xs, W1)
    acc1 = _sc_conv(u1, src, dst, norm, chan_split=False)
    u2 = _tc_mm2(acc1, u1, b1.reshape(1, D), W2)
    acc2 = _sc_conv(u2.reshape(NC * N, D), src, dst, norm, chan_split=True)
    h2 = _tc_h2(acc2, u2, b2.reshape(1, 256))
    out15 = _tc_heads(h2.reshape(1, M), Wadv, Wv1,
                      badv.reshape(1, A), bv1.reshape(1, 64),
                      Wv2, bv2.reshape(1, 64), Wv3, bv3.reshape(1, 1))
    return out15.reshape(1, 3, 5)
